# baseline (speedup 1.0000x reference)
import functools

import jax
import jax.numpy as jnp
from jax import lax
from jax.experimental import pallas as pl
from jax.experimental.pallas import tpu as pltpu
from jax.experimental.pallas import tpu_sc as plsc

NUM_CLASS = 1000000
BATCH = 16384
EMBD = 64
ZD = 128
OUT_D = EMBD + ZD
SPLIT = 500224
TR_BLK = 512
IDX_CHUNK = 128


def _tr_body(a_ref, b_ref, o_ref):
    o_ref[...] = jnp.concatenate([a_ref[...].T, b_ref[...].T], axis=1)


@functools.cache
def _transpose_pack_tc():
    grid = SPLIT // TR_BLK
    return pl.pallas_call(
        _tr_body,
        grid=(grid,),
        in_specs=[
            pl.BlockSpec((EMBD, TR_BLK), lambda i: (0, i)),
            pl.BlockSpec((EMBD, TR_BLK), lambda i: (0, SPLIT // TR_BLK + i)),
        ],
        out_specs=pl.BlockSpec((TR_BLK, 2 * EMBD), lambda i: (i, 0)),
        out_shape=jax.ShapeDtypeStruct((SPLIT, 2 * EMBD), jnp.float32),
    )


@functools.cache
def _sc_gather_concat():
    mesh = plsc.VectorSubcoreMesh(core_axis_name="c", subcore_axis_name="s")
    nw = mesh.num_cores * mesh.num_subcores
    b_per_w = BATCH // nw
    n_chunks = b_per_w // IDX_CHUNK

    @functools.partial(
        pl.kernel,
        out_type=jax.ShapeDtypeStruct((BATCH, OUT_D), jnp.float32),
        mesh=mesh,
        scratch_types=[
            pltpu.VMEM((n_chunks, IDX_CHUNK), jnp.int32),
            pltpu.VMEM((b_per_w, EMBD), jnp.float32),
            pltpu.VMEM((b_per_w, ZD), jnp.float32),
            pltpu.SemaphoreType.DMA,
            pltpu.SemaphoreType.DMA,
        ],
        compiler_params=pltpu.CompilerParams(use_tc_tiling_on_sc=False),
    )
    def k(idx_hbm, z_hbm, t2_hbm, out_hbm, idx_v, rows_v, z_v, gsem, zsem):
        wid = lax.axis_index("s") * mesh.num_cores + lax.axis_index("c")
        base = wid * b_per_w
        pltpu.sync_copy(idx_hbm.at[pl.ds(wid * n_chunks, n_chunks)], idx_v)
        gathers = []
        for j in range(n_chunks):
            gathers.append(pltpu.async_copy(
                t2_hbm.at[idx_v.at[j]],
                rows_v.at[pl.ds(j * IDX_CHUNK, IDX_CHUNK)],
                gsem,
            ))
        zread = pltpu.async_copy(z_hbm.at[pl.ds(base, b_per_w)], z_v, zsem)
        zread.wait()
        zwrite = pltpu.async_copy(
            z_v, out_hbm.at[pl.ds(base, b_per_w), pl.ds(EMBD, ZD)], zsem)
        for g in gathers:
            g.wait()
        pltpu.sync_copy(rows_v, out_hbm.at[pl.ds(base, b_per_w), pl.ds(0, EMBD)])
        zwrite.wait()

    return k


def _pos_body(pos_ref, w_ref, b_ref, out_ref):
    out_ref[...] = lax.dot_general(
        pos_ref[...], w_ref[...], (((1,), (1,)), ((), ())),
        preferred_element_type=jnp.float32,
    ) + b_ref[...]


@functools.cache
def _pos_linear():
    blk = 2048
    grid = BATCH // blk
    return pl.pallas_call(
        _pos_body,
        grid=(grid,),
        in_specs=[
            pl.BlockSpec((blk, 4), lambda i: (i, 0)),
            pl.BlockSpec((EMBD, 4), lambda i: (0, 0)),
            pl.BlockSpec((1, EMBD), lambda i: (0, 0)),
        ],
        out_specs=pl.BlockSpec((blk, EMBD), lambda i: (i, 0)),
        out_shape=jax.ShapeDtypeStruct((BATCH, EMBD), jnp.float32),
    )


def kernel(cla, pos, z, table, W, b):
    t2 = _transpose_pack_tc()(table.T, table.T)
    t2v = t2.reshape(2 * SPLIT, EMBD)
    w = jnp.where(cla < SPLIT, 2 * cla, 2 * (cla - SPLIT) + 1)
    w2d = w.reshape(BATCH // IDX_CHUNK, IDX_CHUNK)
    out = _sc_gather_concat()(w2d, z, t2v)
    pos_embd = _pos_linear()(pos, W, b.reshape(1, EMBD))
    return (out, pos_embd)


# speedup vs baseline: 1.8765x; 1.8765x over previous
import functools

import jax
import jax.numpy as jnp
from jax import lax
from jax.experimental import pallas as pl
from jax.experimental.pallas import tpu as pltpu
from jax.experimental.pallas import tpu_sc as plsc

NUM_CLASS = 1000000
BATCH = 16384
EMBD = 64
ZD = 128
OUT_D = EMBD + ZD
SPLIT = 500480
TR_BLK = 1280
IDX_CHUNK = 128


def _tr_body(a_ref, b_ref, o_ref):
    o_ref[...] = jnp.concatenate([a_ref[...], b_ref[...]], axis=0).T


@functools.cache
def _transpose_pack_tc():
    grid = SPLIT // TR_BLK
    return pl.pallas_call(
        _tr_body,
        grid=(grid,),
        in_specs=[
            pl.BlockSpec((EMBD, TR_BLK), lambda i: (0, i)),
            pl.BlockSpec((EMBD, TR_BLK), lambda i: (0, SPLIT // TR_BLK + i)),
        ],
        out_specs=pl.BlockSpec((TR_BLK, 2 * EMBD), lambda i: (i, 0)),
        out_shape=jax.ShapeDtypeStruct((SPLIT, 2 * EMBD), jnp.float32),
    )


@functools.cache
def _sc_gather_concat():
    mesh = plsc.VectorSubcoreMesh(core_axis_name="c", subcore_axis_name="s")
    nw = mesh.num_cores * mesh.num_subcores
    b_per_w = BATCH // nw
    n_chunks = b_per_w // IDX_CHUNK

    @functools.partial(
        pl.kernel,
        out_type=jax.ShapeDtypeStruct((BATCH, OUT_D), jnp.float32),
        mesh=mesh,
        scratch_types=[
            pltpu.VMEM((n_chunks, IDX_CHUNK), jnp.int32),
            pltpu.VMEM((b_per_w, EMBD), jnp.float32),
            pltpu.VMEM((b_per_w, ZD), jnp.float32),
            pltpu.SemaphoreType.DMA,
            pltpu.SemaphoreType.DMA,
        ],
        compiler_params=pltpu.CompilerParams(use_tc_tiling_on_sc=False),
    )
    def k(idx_hbm, z_hbm, t2_hbm, out_hbm, idx_v, rows_v, z_v, gsem, zsem):
        wid = lax.axis_index("s") * mesh.num_cores + lax.axis_index("c")
        base = wid * b_per_w
        pltpu.sync_copy(idx_hbm.at[pl.ds(wid * n_chunks, n_chunks)], idx_v)
        gathers = []
        for j in range(n_chunks):
            gathers.append(pltpu.async_copy(
                t2_hbm.at[idx_v.at[j]],
                rows_v.at[pl.ds(j * IDX_CHUNK, IDX_CHUNK)],
                gsem,
            ))
        zread = pltpu.async_copy(z_hbm.at[pl.ds(base, b_per_w)], z_v, zsem)
        zread.wait()
        zwrite = pltpu.async_copy(
            z_v, out_hbm.at[pl.ds(base, b_per_w), pl.ds(EMBD, ZD)], zsem)
        for g in gathers:
            g.wait()
        pltpu.sync_copy(rows_v, out_hbm.at[pl.ds(base, b_per_w), pl.ds(0, EMBD)])
        zwrite.wait()

    return k


def _pos_body(pos_ref, w_ref, b_ref, out_ref):
    out_ref[...] = lax.dot_general(
        pos_ref[...], w_ref[...], (((1,), (1,)), ((), ())),
        preferred_element_type=jnp.float32,
    ) + b_ref[...]


@functools.cache
def _pos_linear():
    blk = 2048
    grid = BATCH // blk
    return pl.pallas_call(
        _pos_body,
        grid=(grid,),
        in_specs=[
            pl.BlockSpec((blk, 4), lambda i: (i, 0)),
            pl.BlockSpec((EMBD, 4), lambda i: (0, 0)),
            pl.BlockSpec((1, EMBD), lambda i: (0, 0)),
        ],
        out_specs=pl.BlockSpec((blk, EMBD), lambda i: (i, 0)),
        out_shape=jax.ShapeDtypeStruct((BATCH, EMBD), jnp.float32),
    )


def kernel(cla, pos, z, table, W, b):
    t2 = _transpose_pack_tc()(table.T, table.T)
    t2v = t2.reshape(2 * SPLIT, EMBD)
    w = jnp.where(cla < SPLIT, 2 * cla, 2 * (cla - SPLIT) + 1)
    w2d = w.reshape(BATCH // IDX_CHUNK, IDX_CHUNK)
    out = _sc_gather_concat()(w2d, z, t2v)
    pos_embd = _pos_linear()(pos, W, b.reshape(1, EMBD))
    return (out, pos_embd)


# speedup vs baseline: 1.8785x; 1.0011x over previous
import functools

import jax
import jax.numpy as jnp
from jax import lax
from jax.experimental import pallas as pl
from jax.experimental.pallas import tpu as pltpu
from jax.experimental.pallas import tpu_sc as plsc

NUM_CLASS = 1000000
BATCH = 16384
EMBD = 64
ZD = 128
OUT_D = EMBD + ZD
SPLIT = 500480
TR_BLK = 1280
IDX_CHUNK = 128


def _tr_body(a_ref, b_ref, o_ref):
    o_ref[...] = jnp.concatenate([a_ref[...], b_ref[...]], axis=0).T


@functools.cache
def _transpose_pack_tc():
    grid = SPLIT // TR_BLK
    return pl.pallas_call(
        _tr_body,
        grid=(grid,),
        in_specs=[
            pl.BlockSpec((EMBD, TR_BLK), lambda i: (0, i)),
            pl.BlockSpec((EMBD, TR_BLK), lambda i: (0, SPLIT // TR_BLK + i)),
        ],
        out_specs=pl.BlockSpec((TR_BLK, 2 * EMBD), lambda i: (i, 0)),
        out_shape=jax.ShapeDtypeStruct((SPLIT, 2 * EMBD), jnp.float32),
        compiler_params=pltpu.CompilerParams(
            dimension_semantics=("parallel",)),
    )


@functools.cache
def _sc_gather_concat():
    mesh = plsc.VectorSubcoreMesh(core_axis_name="c", subcore_axis_name="s")
    nw = mesh.num_cores * mesh.num_subcores
    b_per_w = BATCH // nw
    n_chunks = b_per_w // IDX_CHUNK

    @functools.partial(
        pl.kernel,
        out_type=jax.ShapeDtypeStruct((BATCH, OUT_D), jnp.float32),
        mesh=mesh,
        scratch_types=[
            pltpu.VMEM((n_chunks, IDX_CHUNK), jnp.int32),
            pltpu.VMEM((b_per_w, EMBD), jnp.float32),
            pltpu.VMEM((b_per_w, ZD), jnp.float32),
            pltpu.SemaphoreType.DMA,
            pltpu.SemaphoreType.DMA,
        ],
        compiler_params=pltpu.CompilerParams(use_tc_tiling_on_sc=False),
    )
    def k(idx_hbm, z_hbm, t2_hbm, out_hbm, idx_v, rows_v, z_v, gsem, zsem):
        wid = lax.axis_index("s") * mesh.num_cores + lax.axis_index("c")
        base = wid * b_per_w
        pltpu.sync_copy(idx_hbm.at[pl.ds(wid * n_chunks, n_chunks)], idx_v)
        gathers = []
        for j in range(n_chunks):
            gathers.append(pltpu.async_copy(
                t2_hbm.at[idx_v.at[j]],
                rows_v.at[pl.ds(j * IDX_CHUNK, IDX_CHUNK)],
                gsem,
            ))
        zread = pltpu.async_copy(z_hbm.at[pl.ds(base, b_per_w)], z_v, zsem)
        zread.wait()
        zwrite = pltpu.async_copy(
            z_v, out_hbm.at[pl.ds(base, b_per_w), pl.ds(EMBD, ZD)], zsem)
        for g in gathers:
            g.wait()
        pltpu.sync_copy(rows_v, out_hbm.at[pl.ds(base, b_per_w), pl.ds(0, EMBD)])
        zwrite.wait()

    return k


def _pos_body(pos_ref, w_ref, b_ref, out_ref):
    out_ref[...] = lax.dot_general(
        pos_ref[...], w_ref[...], (((1,), (1,)), ((), ())),
        preferred_element_type=jnp.float32,
    ) + b_ref[...]


@functools.cache
def _pos_linear():
    blk = 2048
    grid = BATCH // blk
    return pl.pallas_call(
        _pos_body,
        grid=(grid,),
        in_specs=[
            pl.BlockSpec((blk, 4), lambda i: (i, 0)),
            pl.BlockSpec((EMBD, 4), lambda i: (0, 0)),
            pl.BlockSpec((1, EMBD), lambda i: (0, 0)),
        ],
        out_specs=pl.BlockSpec((blk, EMBD), lambda i: (i, 0)),
        out_shape=jax.ShapeDtypeStruct((BATCH, EMBD), jnp.float32),
    )


def kernel(cla, pos, z, table, W, b):
    t2 = _transpose_pack_tc()(table.T, table.T)
    t2v = t2.reshape(2 * SPLIT, EMBD)
    w = jnp.where(cla < SPLIT, 2 * cla, 2 * (cla - SPLIT) + 1)
    w2d = w.reshape(BATCH // IDX_CHUNK, IDX_CHUNK)
    out = _sc_gather_concat()(w2d, z, t2v)
    pos_embd = _pos_linear()(pos, W, b.reshape(1, EMBD))
    return (out, pos_embd)


# speedup vs baseline: 2.4454x; 1.3018x over previous
import functools

import jax
import jax.numpy as jnp
from jax import lax
from jax.experimental import pallas as pl
from jax.experimental.pallas import tpu as pltpu
from jax.experimental.pallas import tpu_sc as plsc

NUM_CLASS = 1000000
BATCH = 16384
EMBD = 64
ZD = 128
OUT_D = EMBD + ZD
SPLIT = 501760
TR_BLK = 2560
IDX_CHUNK = 128


def _tr_body(a_ref, b_ref, o_ref):
    o_ref[...] = jnp.concatenate([a_ref[...], b_ref[...]], axis=0).T


@functools.cache
def _transpose_pack_tc():
    grid = SPLIT // TR_BLK
    return pl.pallas_call(
        _tr_body,
        grid=(grid,),
        in_specs=[
            pl.BlockSpec((EMBD, TR_BLK), lambda i: (0, i)),
            pl.BlockSpec((EMBD, TR_BLK),
                         lambda i: (0, jnp.minimum(
                             SPLIT // TR_BLK + i,
                             (NUM_CLASS - 1) // TR_BLK))),
        ],
        out_specs=pl.BlockSpec((TR_BLK, 2 * EMBD), lambda i: (i, 0)),
        out_shape=jax.ShapeDtypeStruct((SPLIT, 2 * EMBD), jnp.float32),
    )


@functools.cache
def _sc_gather_concat():
    mesh = plsc.VectorSubcoreMesh(core_axis_name="c", subcore_axis_name="s")
    nw = mesh.num_cores * mesh.num_subcores
    b_per_w = BATCH // nw
    n_chunks = b_per_w // IDX_CHUNK

    @functools.partial(
        pl.kernel,
        out_type=jax.ShapeDtypeStruct((BATCH, OUT_D), jnp.float32),
        mesh=mesh,
        scratch_types=[
            pltpu.VMEM((n_chunks, IDX_CHUNK), jnp.int32),
            pltpu.VMEM((b_per_w, EMBD), jnp.float32),
            pltpu.VMEM((b_per_w, ZD), jnp.float32),
            pltpu.SemaphoreType.DMA,
            pltpu.SemaphoreType.DMA,
        ],
        compiler_params=pltpu.CompilerParams(use_tc_tiling_on_sc=False),
    )
    def k(idx_hbm, z_hbm, t2_hbm, out_hbm, idx_v, rows_v, z_v, gsem, zsem):
        wid = lax.axis_index("s") * mesh.num_cores + lax.axis_index("c")
        base = wid * b_per_w
        pltpu.sync_copy(idx_hbm.at[pl.ds(wid * n_chunks, n_chunks)], idx_v)
        gathers = []
        for j in range(n_chunks):
            gathers.append(pltpu.async_copy(
                t2_hbm.at[idx_v.at[j]],
                rows_v.at[pl.ds(j * IDX_CHUNK, IDX_CHUNK)],
                gsem,
            ))
        zread = pltpu.async_copy(z_hbm.at[pl.ds(base, b_per_w)], z_v, zsem)
        zread.wait()
        zwrite = pltpu.async_copy(
            z_v, out_hbm.at[pl.ds(base, b_per_w), pl.ds(EMBD, ZD)], zsem)
        for g in gathers:
            g.wait()
        pltpu.sync_copy(rows_v, out_hbm.at[pl.ds(base, b_per_w), pl.ds(0, EMBD)])
        zwrite.wait()

    return k


def _pos_body(pos_ref, w_ref, b_ref, out_ref):
    out_ref[...] = lax.dot_general(
        pos_ref[...], w_ref[...], (((1,), (1,)), ((), ())),
        preferred_element_type=jnp.float32,
    ) + b_ref[...]


@functools.cache
def _pos_linear():
    blk = 2048
    grid = BATCH // blk
    return pl.pallas_call(
        _pos_body,
        grid=(grid,),
        in_specs=[
            pl.BlockSpec((blk, 4), lambda i: (i, 0)),
            pl.BlockSpec((EMBD, 4), lambda i: (0, 0)),
            pl.BlockSpec((1, EMBD), lambda i: (0, 0)),
        ],
        out_specs=pl.BlockSpec((blk, EMBD), lambda i: (i, 0)),
        out_shape=jax.ShapeDtypeStruct((BATCH, EMBD), jnp.float32),
    )


def kernel(cla, pos, z, table, W, b):
    t2 = _transpose_pack_tc()(table.T, table.T)
    t2v = t2.reshape(2 * SPLIT, EMBD)
    w = jnp.where(cla < SPLIT, 2 * cla, 2 * (cla - SPLIT) + 1)
    w2d = w.reshape(BATCH // IDX_CHUNK, IDX_CHUNK)
    out = _sc_gather_concat()(w2d, z, t2v)
    pos_embd = _pos_linear()(pos, W, b.reshape(1, EMBD))
    return (out, pos_embd)


# speedup vs baseline: 3.0156x; 1.2332x over previous
import functools

import jax
import jax.numpy as jnp
from jax import lax
from jax.experimental import pallas as pl
from jax.experimental.pallas import tpu as pltpu
from jax.experimental.pallas import tpu_sc as plsc

NUM_CLASS = 1000000
BATCH = 16384
EMBD = 64
ZD = 128
OUT_D = EMBD + ZD
SPLIT = 501760
TR_BLK = 5120
IDX_CHUNK = 128


def _tr_body(a_ref, b_ref, o_ref):
    o_ref[...] = jnp.concatenate([a_ref[...], b_ref[...]], axis=0).T


@functools.cache
def _transpose_pack_tc():
    grid = SPLIT // TR_BLK
    return pl.pallas_call(
        _tr_body,
        grid=(grid,),
        in_specs=[
            pl.BlockSpec((EMBD, TR_BLK), lambda i: (0, i)),
            pl.BlockSpec((EMBD, TR_BLK),
                         lambda i: (0, jnp.minimum(
                             SPLIT // TR_BLK + i,
                             (NUM_CLASS - 1) // TR_BLK))),
        ],
        out_specs=pl.BlockSpec((TR_BLK, 2 * EMBD), lambda i: (i, 0)),
        out_shape=jax.ShapeDtypeStruct((SPLIT, 2 * EMBD), jnp.float32),
    )


@functools.cache
def _sc_gather_concat():
    mesh = plsc.VectorSubcoreMesh(core_axis_name="c", subcore_axis_name="s")
    nw = mesh.num_cores * mesh.num_subcores
    b_per_w = BATCH // nw
    n_chunks = b_per_w // IDX_CHUNK

    @functools.partial(
        pl.kernel,
        out_type=jax.ShapeDtypeStruct((BATCH, OUT_D), jnp.float32),
        mesh=mesh,
        scratch_types=[
            pltpu.VMEM((n_chunks, IDX_CHUNK), jnp.int32),
            pltpu.VMEM((b_per_w, EMBD), jnp.float32),
            pltpu.VMEM((b_per_w, ZD), jnp.float32),
            pltpu.SemaphoreType.DMA,
            pltpu.SemaphoreType.DMA,
        ],
        compiler_params=pltpu.CompilerParams(use_tc_tiling_on_sc=False),
    )
    def k(idx_hbm, z_hbm, t2_hbm, out_hbm, idx_v, rows_v, z_v, gsem, zsem):
        wid = lax.axis_index("s") * mesh.num_cores + lax.axis_index("c")
        base = wid * b_per_w
        pltpu.sync_copy(idx_hbm.at[pl.ds(wid * n_chunks, n_chunks)], idx_v)
        gathers = []
        for j in range(n_chunks):
            gathers.append(pltpu.async_copy(
                t2_hbm.at[idx_v.at[j]],
                rows_v.at[pl.ds(j * IDX_CHUNK, IDX_CHUNK)],
                gsem,
            ))
        zread = pltpu.async_copy(z_hbm.at[pl.ds(base, b_per_w)], z_v, zsem)
        zread.wait()
        zwrite = pltpu.async_copy(
            z_v, out_hbm.at[pl.ds(base, b_per_w), pl.ds(EMBD, ZD)], zsem)
        for g in gathers:
            g.wait()
        pltpu.sync_copy(rows_v, out_hbm.at[pl.ds(base, b_per_w), pl.ds(0, EMBD)])
        zwrite.wait()

    return k


def _pos_body(pos_ref, w_ref, b_ref, out_ref):
    out_ref[...] = lax.dot_general(
        pos_ref[...], w_ref[...], (((1,), (1,)), ((), ())),
        preferred_element_type=jnp.float32,
    ) + b_ref[...]


@functools.cache
def _pos_linear():
    blk = 2048
    grid = BATCH // blk
    return pl.pallas_call(
        _pos_body,
        grid=(grid,),
        in_specs=[
            pl.BlockSpec((blk, 4), lambda i: (i, 0)),
            pl.BlockSpec((EMBD, 4), lambda i: (0, 0)),
            pl.BlockSpec((1, EMBD), lambda i: (0, 0)),
        ],
        out_specs=pl.BlockSpec((blk, EMBD), lambda i: (i, 0)),
        out_shape=jax.ShapeDtypeStruct((BATCH, EMBD), jnp.float32),
    )


def kernel(cla, pos, z, table, W, b):
    t2 = _transpose_pack_tc()(table.T, table.T)
    t2v = t2.reshape(2 * SPLIT, EMBD)
    w = jnp.where(cla < SPLIT, 2 * cla, 2 * (cla - SPLIT) + 1)
    w2d = w.reshape(BATCH // IDX_CHUNK, IDX_CHUNK)
    out = _sc_gather_concat()(w2d, z, t2v)
    pos_embd = _pos_linear()(pos, W, b.reshape(1, EMBD))
    return (out, pos_embd)


# speedup vs baseline: 3.2246x; 1.0693x over previous
import functools

import jax
import jax.numpy as jnp
from jax import lax
from jax.experimental import pallas as pl
from jax.experimental.pallas import tpu as pltpu
from jax.experimental.pallas import tpu_sc as plsc

NUM_CLASS = 1000000
BATCH = 16384
EMBD = 64
ZD = 128
OUT_D = EMBD + ZD
SPLIT = 501760
TR_BLK = 10240
IDX_CHUNK = 128


def _tr_body(a_ref, b_ref, o_ref):
    o_ref[...] = jnp.concatenate([a_ref[...], b_ref[...]], axis=0).T


@functools.cache
def _transpose_pack_tc():
    grid = SPLIT // TR_BLK
    return pl.pallas_call(
        _tr_body,
        grid=(grid,),
        in_specs=[
            pl.BlockSpec((EMBD, TR_BLK), lambda i: (0, i)),
            pl.BlockSpec((EMBD, TR_BLK),
                         lambda i: (0, jnp.minimum(
                             SPLIT // TR_BLK + i,
                             (NUM_CLASS - 1) // TR_BLK))),
        ],
        out_specs=pl.BlockSpec((TR_BLK, 2 * EMBD), lambda i: (i, 0)),
        out_shape=jax.ShapeDtypeStruct((SPLIT, 2 * EMBD), jnp.float32),
    )


@functools.cache
def _sc_gather_concat():
    mesh = plsc.VectorSubcoreMesh(core_axis_name="c", subcore_axis_name="s")
    nw = mesh.num_cores * mesh.num_subcores
    b_per_w = BATCH // nw
    n_chunks = b_per_w // IDX_CHUNK

    @functools.partial(
        pl.kernel,
        out_type=jax.ShapeDtypeStruct((BATCH, OUT_D), jnp.float32),
        mesh=mesh,
        scratch_types=[
            pltpu.VMEM((n_chunks, IDX_CHUNK), jnp.int32),
            pltpu.VMEM((b_per_w, EMBD), jnp.float32),
            pltpu.VMEM((b_per_w, ZD), jnp.float32),
            pltpu.SemaphoreType.DMA,
            pltpu.SemaphoreType.DMA,
        ],
        compiler_params=pltpu.CompilerParams(use_tc_tiling_on_sc=False),
    )
    def k(idx_hbm, z_hbm, t2_hbm, out_hbm, idx_v, rows_v, z_v, gsem, zsem):
        wid = lax.axis_index("s") * mesh.num_cores + lax.axis_index("c")
        base = wid * b_per_w
        pltpu.sync_copy(idx_hbm.at[pl.ds(wid * n_chunks, n_chunks)], idx_v)
        gathers = []
        for j in range(n_chunks):
            gathers.append(pltpu.async_copy(
                t2_hbm.at[idx_v.at[j]],
                rows_v.at[pl.ds(j * IDX_CHUNK, IDX_CHUNK)],
                gsem,
            ))
        zread = pltpu.async_copy(z_hbm.at[pl.ds(base, b_per_w)], z_v, zsem)
        zread.wait()
        zwrite = pltpu.async_copy(
            z_v, out_hbm.at[pl.ds(base, b_per_w), pl.ds(EMBD, ZD)], zsem)
        for g in gathers:
            g.wait()
        pltpu.sync_copy(rows_v, out_hbm.at[pl.ds(base, b_per_w), pl.ds(0, EMBD)])
        zwrite.wait()

    return k


def _pos_body(pos_ref, w_ref, b_ref, out_ref):
    out_ref[...] = lax.dot_general(
        pos_ref[...], w_ref[...], (((1,), (1,)), ((), ())),
        preferred_element_type=jnp.float32,
    ) + b_ref[...]


@functools.cache
def _pos_linear():
    blk = 2048
    grid = BATCH // blk
    return pl.pallas_call(
        _pos_body,
        grid=(grid,),
        in_specs=[
            pl.BlockSpec((blk, 4), lambda i: (i, 0)),
            pl.BlockSpec((EMBD, 4), lambda i: (0, 0)),
            pl.BlockSpec((1, EMBD), lambda i: (0, 0)),
        ],
        out_specs=pl.BlockSpec((blk, EMBD), lambda i: (i, 0)),
        out_shape=jax.ShapeDtypeStruct((BATCH, EMBD), jnp.float32),
    )


def kernel(cla, pos, z, table, W, b):
    t2 = _transpose_pack_tc()(table.T, table.T)
    t2v = t2.reshape(2 * SPLIT, EMBD)
    w = jnp.where(cla < SPLIT, 2 * cla, 2 * (cla - SPLIT) + 1)
    w2d = w.reshape(BATCH // IDX_CHUNK, IDX_CHUNK)
    out = _sc_gather_concat()(w2d, z, t2v)
    pos_embd = _pos_linear()(pos, W, b.reshape(1, EMBD))
    return (out, pos_embd)


# speedup vs baseline: 3.2551x; 1.0095x over previous
import functools

import jax
import jax.numpy as jnp
from jax import lax
from jax.experimental import pallas as pl
from jax.experimental.pallas import tpu as pltpu
from jax.experimental.pallas import tpu_sc as plsc

NUM_CLASS = 1000000
BATCH = 16384
EMBD = 64
ZD = 128
OUT_D = EMBD + ZD
SPLIT = 512000
TR_BLK = 20480
IDX_CHUNK = 128


def _tr_body(a_ref, b_ref, o_ref):
    o_ref[...] = jnp.concatenate([a_ref[...], b_ref[...]], axis=0).T


@functools.cache
def _transpose_pack_tc():
    grid = SPLIT // TR_BLK
    return pl.pallas_call(
        _tr_body,
        grid=(grid,),
        in_specs=[
            pl.BlockSpec((EMBD, TR_BLK), lambda i: (0, i)),
            pl.BlockSpec((EMBD, TR_BLK),
                         lambda i: (0, jnp.minimum(
                             SPLIT // TR_BLK + i,
                             (NUM_CLASS - 1) // TR_BLK))),
        ],
        out_specs=pl.BlockSpec((TR_BLK, 2 * EMBD), lambda i: (i, 0)),
        out_shape=jax.ShapeDtypeStruct((SPLIT, 2 * EMBD), jnp.float32),
        compiler_params=pltpu.CompilerParams(
            vmem_limit_bytes=48 * 1024 * 1024),
    )


@functools.cache
def _sc_gather_concat():
    mesh = plsc.VectorSubcoreMesh(core_axis_name="c", subcore_axis_name="s")
    nw = mesh.num_cores * mesh.num_subcores
    b_per_w = BATCH // nw
    n_chunks = b_per_w // IDX_CHUNK

    @functools.partial(
        pl.kernel,
        out_type=jax.ShapeDtypeStruct((BATCH, OUT_D), jnp.float32),
        mesh=mesh,
        scratch_types=[
            pltpu.VMEM((n_chunks, IDX_CHUNK), jnp.int32),
            pltpu.VMEM((b_per_w, EMBD), jnp.float32),
            pltpu.VMEM((b_per_w, ZD), jnp.float32),
            pltpu.SemaphoreType.DMA,
            pltpu.SemaphoreType.DMA,
        ],
        compiler_params=pltpu.CompilerParams(use_tc_tiling_on_sc=False),
    )
    def k(idx_hbm, z_hbm, t2_hbm, out_hbm, idx_v, rows_v, z_v, gsem, zsem):
        wid = lax.axis_index("s") * mesh.num_cores + lax.axis_index("c")
        base = wid * b_per_w
        pltpu.sync_copy(idx_hbm.at[pl.ds(wid * n_chunks, n_chunks)], idx_v)
        gathers = []
        for j in range(n_chunks):
            gathers.append(pltpu.async_copy(
                t2_hbm.at[idx_v.at[j]],
                rows_v.at[pl.ds(j * IDX_CHUNK, IDX_CHUNK)],
                gsem,
            ))
        zread = pltpu.async_copy(z_hbm.at[pl.ds(base, b_per_w)], z_v, zsem)
        zread.wait()
        zwrite = pltpu.async_copy(
            z_v, out_hbm.at[pl.ds(base, b_per_w), pl.ds(EMBD, ZD)], zsem)
        for g in gathers:
            g.wait()
        pltpu.sync_copy(rows_v, out_hbm.at[pl.ds(base, b_per_w), pl.ds(0, EMBD)])
        zwrite.wait()

    return k


def _pos_body(pos_ref, w_ref, b_ref, out_ref):
    out_ref[...] = lax.dot_general(
        pos_ref[...], w_ref[...], (((1,), (1,)), ((), ())),
        preferred_element_type=jnp.float32,
    ) + b_ref[...]


@functools.cache
def _pos_linear():
    blk = 2048
    grid = BATCH // blk
    return pl.pallas_call(
        _pos_body,
        grid=(grid,),
        in_specs=[
            pl.BlockSpec((blk, 4), lambda i: (i, 0)),
            pl.BlockSpec((EMBD, 4), lambda i: (0, 0)),
            pl.BlockSpec((1, EMBD), lambda i: (0, 0)),
        ],
        out_specs=pl.BlockSpec((blk, EMBD), lambda i: (i, 0)),
        out_shape=jax.ShapeDtypeStruct((BATCH, EMBD), jnp.float32),
    )


def kernel(cla, pos, z, table, W, b):
    t2 = _transpose_pack_tc()(table.T, table.T)
    t2v = t2.reshape(2 * SPLIT, EMBD)
    w = jnp.where(cla < SPLIT, 2 * cla, 2 * (cla - SPLIT) + 1)
    w2d = w.reshape(BATCH // IDX_CHUNK, IDX_CHUNK)
    out = _sc_gather_concat()(w2d, z, t2v)
    pos_embd = _pos_linear()(pos, W, b.reshape(1, EMBD))
    return (out, pos_embd)
